# baseline (speedup 1.0000x reference)
import jax
import jax.numpy as jnp
from jax import lax
from jax.experimental import pallas as pl
from jax.experimental.pallas import tpu as pltpu
from jax.experimental.pallas import tpu_sc as plsc

LANES = 16
BINS = 256
CHUNK = 8192


def _sc_body(x_hbm, out_hbm, buf0, buf1, hist, merged, pbuf, outbuf0, outbuf1,
             shared, sem0, sem1, osem):
    c = lax.axis_index("c")
    s = lax.axis_index("s")
    row = 8 * c + (s // 2)
    half = s % 2
    n = x_hbm.shape[1]
    half_n = n // 2
    nchunk = half_n // CHUNK
    base = half * half_n

    zeros = jnp.zeros((LANES,), jnp.float32)

    @plsc.parallel_loop(0, BINS, unroll=8)
    def _(i):
        hist[i, :] = zeros

    @plsc.parallel_loop(0, BINS // LANES, unroll=4)
    def _(i):
        merged[pl.ds(i * LANES, LANES)] = zeros

    lane_off = lax.iota(jnp.int32, LANES)
    ones = jnp.ones((LANES,), jnp.float32)

    def chunk_loop(buf):
        @plsc.parallel_loop(0, CHUNK // LANES, unroll=16)
        def _(i):
            v = buf[pl.ds(i * LANES, LANES)]
            plsc.addupdate_scatter(hist, [v.astype(jnp.int32), lane_off], ones)

    cp = pltpu.async_copy(x_hbm.at[row, pl.ds(base, CHUNK)], buf0, sem0)
    for g in range(nchunk):
        buf, nbuf, nsem = (buf0, buf1, sem1) if g % 2 == 0 else (buf1, buf0, sem0)
        ncp = None
        if g + 1 < nchunk:
            ncp = pltpu.async_copy(
                x_hbm.at[row, pl.ds(base + (g + 1) * CHUNK, CHUNK)], nbuf, nsem
            )
        cp.wait()
        chunk_loop(buf)
        cp = ncp

    @plsc.parallel_loop(0, BINS, unroll=4)
    def _(b):
        idx = lax.broadcast(b, (LANES,))
        plsc.addupdate_scatter(merged, [idx], hist[b, :])

    pltpu.sync_copy(merged, shared.at[s])
    plsc.subcore_barrier()
    pltpu.sync_copy(shared.at[s ^ 1], pbuf)

    @plsc.parallel_loop(0, BINS // LANES, unroll=4)
    def _(i):
        sl = pl.ds(i * LANES, LANES)
        merged[sl] = merged[sl] + pbuf[sl]

    gb0 = half * (BINS // 2)
    ocps = []
    for piece, outbuf in enumerate((outbuf0, outbuf1)):
        pb0 = gb0 + piece * (BINS // 4)

        @plsc.parallel_loop(0, BINS // 4, unroll=2)
        def _(b):
            idx = lax.broadcast(pb0 + b, (LANES,))
            tot = plsc.load_gather(merged, [idx])
            for k in range(BINS // LANES):
                outbuf[b, pl.ds(k * LANES, LANES)] = tot

        ocps.append(
            pltpu.async_copy(outbuf, out_hbm.at[row, pl.ds(pb0, BINS // 4)], osem)
        )
    for ocp in ocps:
        ocp.wait()


@jax.jit
def kernel(x):
    b, n = x.shape
    sc_hist = pl.kernel(
        _sc_body,
        out_type=jax.ShapeDtypeStruct((b, BINS, BINS), jnp.float32),
        mesh=plsc.VectorSubcoreMesh(core_axis_name="c", subcore_axis_name="s"),
        scratch_types=[
            pltpu.VMEM((CHUNK,), jnp.float32),
            pltpu.VMEM((CHUNK,), jnp.float32),
            pltpu.VMEM((BINS, LANES), jnp.float32),
            pltpu.VMEM((BINS,), jnp.float32),
            pltpu.VMEM((BINS,), jnp.float32),
            pltpu.VMEM((BINS // 4, BINS), jnp.float32),
            pltpu.VMEM((BINS // 4, BINS), jnp.float32),
            pltpu.VMEM_SHARED((LANES, BINS), jnp.float32),
            pltpu.SemaphoreType.DMA,
            pltpu.SemaphoreType.DMA,
            pltpu.SemaphoreType.DMA,
        ],
        compiler_params=pltpu.CompilerParams(needs_layout_passes=False),
    )
    return sc_hist(x)


# speedup vs baseline: 1.1504x; 1.1504x over previous
import jax
import jax.numpy as jnp
from jax import lax
from jax.experimental import pallas as pl
from jax.experimental.pallas import tpu as pltpu
from jax.experimental.pallas import tpu_sc as plsc

LANES = 16
BINS = 256
CHUNK = 16384


def _sc_body(x_hbm, out_hbm, buf0, buf1, hist, merged, pbuf, outbuf0, outbuf1,
             shared, sem0, sem1, osem):
    c = lax.axis_index("c")
    s = lax.axis_index("s")
    row = 8 * c + (s // 2)
    half = s % 2
    n = x_hbm.shape[1]
    half_n = n // 2
    nchunk = half_n // CHUNK
    base = half * half_n

    zeros = jnp.zeros((LANES,), jnp.float32)

    @plsc.parallel_loop(0, BINS, unroll=8)
    def _(i):
        hist[i, :] = zeros

    @plsc.parallel_loop(0, BINS // LANES, unroll=4)
    def _(i):
        merged[pl.ds(i * LANES, LANES)] = zeros

    lane_off = lax.iota(jnp.int32, LANES)
    ones = jnp.ones((LANES,), jnp.float32)

    def chunk_loop(buf):
        @plsc.parallel_loop(0, CHUNK // LANES, unroll=16)
        def _(i):
            v = buf[pl.ds(i * LANES, LANES)]
            plsc.addupdate_scatter(hist, [v.astype(jnp.int32), lane_off], ones)

    cp = pltpu.async_copy(x_hbm.at[row, pl.ds(base, CHUNK)], buf0, sem0)
    for g in range(nchunk):
        buf, nbuf, nsem = (buf0, buf1, sem1) if g % 2 == 0 else (buf1, buf0, sem0)
        ncp = None
        if g + 1 < nchunk:
            ncp = pltpu.async_copy(
                x_hbm.at[row, pl.ds(base + (g + 1) * CHUNK, CHUNK)], nbuf, nsem
            )
        cp.wait()
        chunk_loop(buf)
        cp = ncp

    @plsc.parallel_loop(0, BINS, unroll=4)
    def _(b):
        idx = lax.broadcast(b, (LANES,))
        plsc.addupdate_scatter(merged, [idx], hist[b, :])

    pltpu.sync_copy(merged, shared.at[s])
    plsc.subcore_barrier()
    pltpu.sync_copy(shared.at[s ^ 1], pbuf)

    @plsc.parallel_loop(0, BINS // LANES, unroll=4)
    def _(i):
        sl = pl.ds(i * LANES, LANES)
        merged[sl] = merged[sl] + pbuf[sl]

    gb0 = half * (BINS // 2)
    ocps = []
    for piece, outbuf in enumerate((outbuf0, outbuf1)):
        pb0 = gb0 + piece * (BINS // 4)

        @plsc.parallel_loop(0, BINS // 4, unroll=2)
        def _(b):
            idx = lax.broadcast(pb0 + b, (LANES,))
            tot = plsc.load_gather(merged, [idx])
            for k in range(BINS // LANES):
                outbuf[b, pl.ds(k * LANES, LANES)] = tot

        ocps.append(
            pltpu.async_copy(outbuf, out_hbm.at[row, pl.ds(pb0, BINS // 4)], osem)
        )
    for ocp in ocps:
        ocp.wait()


@jax.jit
def kernel(x):
    b, n = x.shape
    sc_hist = pl.kernel(
        _sc_body,
        out_type=jax.ShapeDtypeStruct((b, BINS, BINS), jnp.float32),
        mesh=plsc.VectorSubcoreMesh(core_axis_name="c", subcore_axis_name="s"),
        scratch_types=[
            pltpu.VMEM((CHUNK,), jnp.float32),
            pltpu.VMEM((CHUNK,), jnp.float32),
            pltpu.VMEM((BINS, LANES), jnp.float32),
            pltpu.VMEM((BINS,), jnp.float32),
            pltpu.VMEM((BINS,), jnp.float32),
            pltpu.VMEM((BINS // 4, BINS), jnp.float32),
            pltpu.VMEM((BINS // 4, BINS), jnp.float32),
            pltpu.VMEM_SHARED((LANES, BINS), jnp.float32),
            pltpu.SemaphoreType.DMA,
            pltpu.SemaphoreType.DMA,
            pltpu.SemaphoreType.DMA,
        ],
        compiler_params=pltpu.CompilerParams(needs_layout_passes=False),
    )
    return sc_hist(x)
